# Initial kernel scaffold; baseline (speedup 1.0000x reference)
import jax
import jax.numpy as jnp
from jax.experimental import pallas as pl


def kernel(x, edge_index, edge_attr, orders, batch, w1_0, b1_0, w2_0, b2_0, w1_1, b1_1, w2_1, b2_1, w1_2, b1_2, w2_2, b2_2, wo1, bo1, wo2, bo2, ws1, bs1, ws2, bs2, ws3, bs3):
    raise NotImplementedError("write your pallas kernel here")


# speedup vs baseline: 2.7645x; 2.7645x over previous
import functools
import math

import jax
import jax.numpy as jnp
from jax import lax
from jax.experimental import pallas as pl
from jax.experimental.pallas import tpu as pltpu
from jax.experimental.pallas import tpu_sc as plsc

N = 10000
E = 320000
F = 128
H = 128
B = 64
NO = 16
OD = 32

NPAD = 10240
NACC = 10112
NC = 2
NS = 16
NW = NC * NS
EPC = 128
NCH = 80
SB = 8
NSB = NCH // SB
EPW = EPC * NCH
EPAD = EPW * NW
ZROWS = NACC // NS

NB = 10048
CHP = 32

BNSCALE = 1.0 / math.sqrt(1.0 + 1e-5)

def _gin_aggregate_body(h_hbm, src_hbm, dst_hbm, zeros_hbm, out_hbm,
                        src_sb, dst_sb, rows_v, acc_sh,
                        sem0, sem1, isem_s, isem_d):
    c = lax.axis_index("c")
    s = lax.axis_index("s")
    w = c * NS + s
    pltpu.sync_copy(zeros_hbm, acc_sh.at[pl.ds(s * ZROWS, ZROWS)])
    plsc.subcore_barrier()

    sems = (sem0, sem1)

    def load_ids(k):
        buf = k % 2
        return (pltpu.async_copy(src_hbm.at[w, pl.ds(k * SB, SB)],
                                 src_sb.at[buf], isem_s),
                pltpu.async_copy(dst_hbm.at[w, pl.ds(k * SB, SB)],
                                 dst_sb.at[buf], isem_d))

    def gather(j):
        return pltpu.async_copy(h_hbm.at[src_sb.at[(j // SB) % 2, j % SB]],
                                rows_v.at[j % 2], sems[j % 2])

    for d in load_ids(0):
        d.wait()
    idd = {1: load_ids(1)} if NSB > 1 else {}
    descs = {j: gather(j) for j in range(2)}
    for j in range(NCH):
        descs.pop(j).wait()
        pltpu.sync_copy(rows_v.at[j % 2],
                        acc_sh.at[dst_sb.at[(j // SB) % 2, j % SB]], add=True)
        if j % SB == SB - 1 and j // SB + 2 < NSB:
            idd[j // SB + 2] = load_ids(j // SB + 2)
        if j + 2 < NCH:
            kk = (j + 2) // SB
            if kk in idd:
                for d in idd.pop(kk):
                    d.wait()
            descs[j + 2] = gather(j + 2)

    plsc.subcore_barrier()
    pltpu.sync_copy(acc_sh.at[pl.ds(s * ZROWS, ZROWS)],
                    out_hbm.at[c, pl.ds(s * ZROWS, ZROWS)])


def _pool_body(h_hbm, bid_hbm, mean_hbm, max_hbm, add_hbm,
               bid_v, buf_v, mrow_v, xrow_v, srow_v):
    c = lax.axis_index("c")
    s = lax.axis_index("s")
    w = c * NS + s
    pltpu.sync_copy(bid_hbm, bid_v)

    t0 = 2 * w
    zero16 = jnp.zeros((16,), jnp.int32)

    def cbody(i, accs):
        v = bid_v[pl.ds(i * 16, 16)]
        a0, a1, a2 = accs
        a0 = a0 + jnp.where(v < t0, 1, 0)
        a1 = a1 + jnp.where(v < t0 + 1, 1, 0)
        a2 = a2 + jnp.where(v < t0 + 2, 1, 0)
        return (a0, a1, a2)

    a0, a1, a2 = lax.fori_loop(0, NB // 16, cbody, (zero16, zero16, zero16))

    def lanesum(a):
        sc = a[0]
        for lane in range(1, 16):
            sc = sc + a[lane]
        return sc

    starts = (lanesum(a0), lanesum(a1), lanesum(a2))

    neg = jnp.float32(-jnp.inf)

    for si in range(2):
        start = starts[si]
        end = starts[si + 1]
        cnt = end - start
        abase = pl.multiple_of((start // 8) * 8, 8)
        nch = (end - abase + CHP - 1) // CHP

        def chunk(k, carry):
            maxs, sums = carry
            base = pl.multiple_of(abase + k * CHP, 8)
            pltpu.sync_copy(h_hbm.at[pl.ds(base, CHP)], buf_v)
            maxs = list(maxs)
            sums = list(sums)
            for r in range(CHP):
                row = base + r
                valid = (row >= start) & (row < end)
                for g in range(8):
                    v = buf_v[r, pl.ds(g * 16, 16)]
                    maxs[g] = jnp.maximum(maxs[g], jnp.where(valid, v, neg))
                    sums[g] = sums[g] + jnp.where(valid, v, 0.0)
            return (tuple(maxs), tuple(sums))

        init = (tuple(jnp.full((16,), neg) for _ in range(8)),
                tuple(jnp.zeros((16,), jnp.float32) for _ in range(8)))
        maxs, sums = lax.fori_loop(0, nch, chunk, init)

        cntv = jnp.zeros((16,), jnp.float32) + cnt.astype(jnp.float32)
        recip = 1.0 / jnp.maximum(cntv, 1.0)
        for g in range(8):
            xrow_v[si, pl.ds(g * 16, 16)] = maxs[g]
            srow_v[si, pl.ds(g * 16, 16)] = sums[g]
            mrow_v[si, pl.ds(g * 16, 16)] = sums[g] * recip

    pltpu.sync_copy(mrow_v, mean_hbm.at[w])
    pltpu.sync_copy(xrow_v, max_hbm.at[w])
    pltpu.sync_copy(srow_v, add_hbm.at[w])


@functools.cache
def _sc_kernels():
    mesh = plsc.VectorSubcoreMesh(core_axis_name="c", subcore_axis_name="s",
                                  num_cores=NC, num_subcores=NS)
    gin_aggregate = functools.partial(
        pl.kernel,
        out_type=jax.ShapeDtypeStruct((NC, NPAD, H), jnp.float32),
        mesh=mesh,
        scratch_types=[
            pltpu.VMEM((2, SB, EPC), jnp.int32),
            pltpu.VMEM((2, SB, EPC), jnp.int32),
            pltpu.VMEM((2, EPC, H), jnp.float32),
            pltpu.VMEM_SHARED((NACC, H), jnp.float32),
            pltpu.SemaphoreType.DMA,
            pltpu.SemaphoreType.DMA,
            pltpu.SemaphoreType.DMA,
            pltpu.SemaphoreType.DMA,
        ],
    )(_gin_aggregate_body)
    pool = functools.partial(
        pl.kernel,
        out_type=(jax.ShapeDtypeStruct((NW, 2, H), jnp.float32),
                  jax.ShapeDtypeStruct((NW, 2, H), jnp.float32),
                  jax.ShapeDtypeStruct((NW, 2, H), jnp.float32)),
        mesh=mesh,
        scratch_types=[
            pltpu.VMEM((NB,), jnp.int32),
            pltpu.VMEM((CHP, H), jnp.float32),
            pltpu.VMEM((2, H), jnp.float32),
            pltpu.VMEM((2, H), jnp.float32),
            pltpu.VMEM((2, H), jnp.float32),
        ],
    )(_pool_body)
    return gin_aggregate, pool


def _dot(a, b):
    return jnp.dot(a, b, preferred_element_type=jnp.float32,
                   precision=lax.Precision.HIGHEST)


def _mlp_body(h_ref, p0_ref, p1_ref, w1_ref, b1_ref, w2_ref, b2_ref, o_ref,
              *, relu_out):
    m = h_ref[...] + p0_ref[...] + p1_ref[...]
    y = jnp.maximum(_dot(m, w1_ref[...]) + b1_ref[...], 0.0)
    y = (_dot(y, w2_ref[...]) + b2_ref[...]) * BNSCALE
    if relu_out:
        y = jnp.maximum(y, 0.0)
    o_ref[...] = y


def _mlp(h, p, w1, b1, w2, b2, relu_out):
    blk = 1024
    return pl.pallas_call(
        functools.partial(_mlp_body, relu_out=relu_out),
        grid=(NPAD // blk,),
        in_specs=[
            pl.BlockSpec((blk, H), lambda i: (i, 0)),
            pl.BlockSpec((blk, H), lambda i: (i, 0)),
            pl.BlockSpec((blk, H), lambda i: (i, 0)),
            pl.BlockSpec((H, H), lambda i: (0, 0)),
            pl.BlockSpec((1, H), lambda i: (0, 0)),
            pl.BlockSpec((H, H), lambda i: (0, 0)),
            pl.BlockSpec((1, H), lambda i: (0, 0)),
        ],
        out_specs=pl.BlockSpec((blk, H), lambda i: (i, 0)),
        out_shape=jax.ShapeDtypeStruct((NPAD, H), jnp.float32),
    )(h, p[0], p[1], w1, b1.reshape(1, H), w2, b2.reshape(1, H))


def _score_body(mean_ref, max_ref, add_ref, ord_ref,
                wo1_ref, bo1_ref, wo2_ref, bo2_ref,
                wm_ref, wx_ref, wa_ref, we_ref, bs1_ref,
                ws2_ref, bs2_ref, ws3_ref, bs3_ref, o_ref):
    oe = jnp.maximum(_dot(ord_ref[...], wo1_ref[...]) + bo1_ref[...], 0.0)
    oe = _dot(oe, wo2_ref[...]) + bo2_ref[...]
    pp = (_dot(mean_ref[...], wm_ref[...]) + _dot(max_ref[...], wx_ref[...])
          + _dot(add_ref[...], wa_ref[...]) + bs1_ref[...])
    ppb = jnp.broadcast_to(pp[:, None, :], (B, NO, H)).reshape(B * NO, H)
    s1 = jnp.maximum(_dot(oe, we_ref[...]) + ppb, 0.0)
    s2 = jnp.maximum(_dot(s1, ws2_ref[...]) + bs2_ref[...], 0.0)
    o_ref[...] = _dot(s2, ws3_ref[...]) + bs3_ref[...]


def _score(meanp, maxp, addp, ords, wo1, bo1, wo2, bo2, ws1, bs1,
           ws2, bs2, ws3, bs3):
    ws3p = jnp.zeros((H // 2, H), jnp.float32).at[:, 0].set(ws3[:, 0])
    return pl.pallas_call(
        _score_body,
        out_shape=jax.ShapeDtypeStruct((B * NO, H), jnp.float32),
    )(meanp, maxp, addp, ords,
      wo1, bo1.reshape(1, H), wo2, bo2.reshape(1, H),
      ws1[0:H], ws1[H:2 * H], ws1[2 * H:3 * H], ws1[3 * H:4 * H],
      bs1.reshape(1, H), ws2, bs2.reshape(1, H // 2), ws3p,
      bs3.reshape(1, 1))


def kernel(x, edge_index, edge_attr, orders, batch,
           w1_0, b1_0, w2_0, b2_0,
           w1_1, b1_1, w2_1, b2_1,
           w1_2, b1_2, w2_2, b2_2,
           wo1, bo1, wo2, bo2,
           ws1, bs1, ws2, bs2, ws3, bs3):
    pad = EPAD - E
    srcp = jnp.concatenate(
        [edge_index[0], jnp.zeros((pad,), jnp.int32)]).reshape(NW, NCH, EPC)
    dstp = jnp.concatenate(
        [edge_index[1], jnp.full((pad,), N, jnp.int32)]).reshape(NW, NCH, EPC)
    zeros_blk = jnp.zeros((ZROWS, H), jnp.float32)
    h = jnp.zeros((NPAD, F), jnp.float32).at[:N].set(x)
    batch_p = jnp.concatenate([batch, jnp.full((NB - N,), B, jnp.int32)])

    gin_aggregate, pool = _sc_kernels()
    convs = [(w1_0, b1_0, w2_0, b2_0, True),
             (w1_1, b1_1, w2_1, b2_1, True),
             (w1_2, b1_2, w2_2, b2_2, False)]
    for w1, b1, w2, b2, relu_out in convs:
        p = gin_aggregate(h, srcp, dstp, zeros_blk)
        h = _mlp(h, p, w1, b1, w2, b2, relu_out)

    meanp, maxp, addp = pool(h, batch_p)
    meanp = meanp.reshape(B, H)
    maxp = maxp.reshape(B, H)
    addp = addp.reshape(B, H)
    s3 = _score(meanp, maxp, addp, orders.reshape(B * NO, OD),
                wo1, bo1, wo2, bo2, ws1, bs1, ws2, bs2, ws3, bs3)
    return s3[:, 0].reshape(B, NO)


# speedup vs baseline: 2.8122x; 1.0173x over previous
import functools
import math

import jax
import jax.numpy as jnp
from jax import lax
from jax.experimental import pallas as pl
from jax.experimental.pallas import tpu as pltpu
from jax.experimental.pallas import tpu_sc as plsc

N = 10000
E = 320000
F = 128
H = 128
B = 64
NO = 16
OD = 32

NPAD = 10240
NACC = 10112
NC = 2
NS = 16
NW = NC * NS
EPC = 128
NCH = 80
SB = 8
NSB = NCH // SB
EPW = EPC * NCH
EPAD = EPW * NW
ZROWS = NACC // NS

NB = 10048
CHP = 32

BNSCALE = 1.0 / math.sqrt(1.0 + 1e-5)

def _gin_aggregate_body(h_hbm, src_hbm, dst_hbm, zeros_hbm, out_hbm,
                        src_sb, dst_sb, rows_v, acc_sh,
                        sem0, sem1, isem_s, isem_d):
    c = lax.axis_index("c")
    s = lax.axis_index("s")
    w = c * NS + s
    pltpu.sync_copy(zeros_hbm, acc_sh.at[pl.ds(s * ZROWS, ZROWS)])
    plsc.subcore_barrier()

    sems = (sem0, sem1)

    def load_ids(k):
        buf = k % 2
        return (pltpu.async_copy(src_hbm.at[w, pl.ds(k * SB, SB)],
                                 src_sb.at[buf], isem_s),
                pltpu.async_copy(dst_hbm.at[w, pl.ds(k * SB, SB)],
                                 dst_sb.at[buf], isem_d))

    def gather(j):
        return pltpu.async_copy(h_hbm.at[src_sb.at[(j // SB) % 2, j % SB]],
                                rows_v.at[j % 2], sems[j % 2])

    for d in load_ids(0):
        d.wait()
    idd = {1: load_ids(1)} if NSB > 1 else {}
    descs = {j: gather(j) for j in range(2)}
    for j in range(NCH):
        descs.pop(j).wait()
        pltpu.sync_copy(rows_v.at[j % 2],
                        acc_sh.at[dst_sb.at[(j // SB) % 2, j % SB]], add=True)
        if j % SB == SB - 1 and j // SB + 2 < NSB:
            idd[j // SB + 2] = load_ids(j // SB + 2)
        if j + 2 < NCH:
            kk = (j + 2) // SB
            if kk in idd:
                for d in idd.pop(kk):
                    d.wait()
            descs[j + 2] = gather(j + 2)

    plsc.subcore_barrier()
    pltpu.sync_copy(acc_sh.at[pl.ds(s * ZROWS, ZROWS)],
                    out_hbm.at[c, pl.ds(s * ZROWS, ZROWS)])


def _pool_body(h_hbm, bid_hbm, mean_hbm, max_hbm, add_hbm,
               bid_v, buf_v, mrow_v, xrow_v, srow_v):
    c = lax.axis_index("c")
    s = lax.axis_index("s")
    w = c * NS + s
    pltpu.sync_copy(bid_hbm, bid_v)

    t0 = 2 * w
    zero16 = jnp.zeros((16,), jnp.int32)

    def cbody(i, accs):
        v = bid_v[pl.ds(i * 16, 16)]
        a0, a1, a2 = accs
        a0 = a0 + jnp.where(v < t0, 1, 0)
        a1 = a1 + jnp.where(v < t0 + 1, 1, 0)
        a2 = a2 + jnp.where(v < t0 + 2, 1, 0)
        return (a0, a1, a2)

    a0, a1, a2 = lax.fori_loop(0, NB // 16, cbody, (zero16, zero16, zero16))

    def lanesum(a):
        sc = a[0]
        for lane in range(1, 16):
            sc = sc + a[lane]
        return sc

    starts = (lanesum(a0), lanesum(a1), lanesum(a2))

    neg = jnp.float32(-jnp.inf)

    for si in range(2):
        start = starts[si]
        end = starts[si + 1]
        cnt = end - start
        abase = pl.multiple_of((start // 8) * 8, 8)
        nch = (end - abase + CHP - 1) // CHP

        def chunk(k, carry):
            maxs, sums = carry
            base = pl.multiple_of(abase + k * CHP, 8)
            pltpu.sync_copy(h_hbm.at[pl.ds(base, CHP)], buf_v)
            maxs = list(maxs)
            sums = list(sums)
            for r in range(CHP):
                row = base + r
                valid = (row >= start) & (row < end)
                for g in range(8):
                    v = buf_v[r, pl.ds(g * 16, 16)]
                    maxs[g] = jnp.maximum(maxs[g], jnp.where(valid, v, neg))
                    sums[g] = sums[g] + jnp.where(valid, v, 0.0)
            return (tuple(maxs), tuple(sums))

        init = (tuple(jnp.full((16,), neg) for _ in range(8)),
                tuple(jnp.zeros((16,), jnp.float32) for _ in range(8)))
        maxs, sums = lax.fori_loop(0, nch, chunk, init)

        cntv = jnp.zeros((16,), jnp.float32) + cnt.astype(jnp.float32)
        recip = 1.0 / jnp.maximum(cntv, 1.0)
        for g in range(8):
            xrow_v[si, pl.ds(g * 16, 16)] = maxs[g]
            srow_v[si, pl.ds(g * 16, 16)] = sums[g]
            mrow_v[si, pl.ds(g * 16, 16)] = sums[g] * recip

    pltpu.sync_copy(mrow_v, mean_hbm.at[w])
    pltpu.sync_copy(xrow_v, max_hbm.at[w])
    pltpu.sync_copy(srow_v, add_hbm.at[w])


@functools.cache
def _sc_kernels():
    mesh = plsc.VectorSubcoreMesh(core_axis_name="c", subcore_axis_name="s",
                                  num_cores=NC, num_subcores=NS)
    gin_aggregate = functools.partial(
        pl.kernel,
        out_type=jax.ShapeDtypeStruct((NC, NPAD, H), jnp.float32),
        mesh=mesh,
        scratch_types=[
            pltpu.VMEM((2, SB, EPC), jnp.int32),
            pltpu.VMEM((2, SB, EPC), jnp.int32),
            pltpu.VMEM((2, EPC, H), jnp.float32),
            pltpu.VMEM_SHARED((NACC, H), jnp.float32),
            pltpu.SemaphoreType.DMA,
            pltpu.SemaphoreType.DMA,
            pltpu.SemaphoreType.DMA,
            pltpu.SemaphoreType.DMA,
        ],
    )(_gin_aggregate_body)
    pool = functools.partial(
        pl.kernel,
        out_type=(jax.ShapeDtypeStruct((NW, 2, H), jnp.float32),
                  jax.ShapeDtypeStruct((NW, 2, H), jnp.float32),
                  jax.ShapeDtypeStruct((NW, 2, H), jnp.float32)),
        mesh=mesh,
        scratch_types=[
            pltpu.VMEM((NB,), jnp.int32),
            pltpu.VMEM((CHP, H), jnp.float32),
            pltpu.VMEM((2, H), jnp.float32),
            pltpu.VMEM((2, H), jnp.float32),
            pltpu.VMEM((2, H), jnp.float32),
        ],
    )(_pool_body)
    return gin_aggregate, pool


def _dot(a, b):
    return jnp.dot(a.astype(jnp.bfloat16), b.astype(jnp.bfloat16),
                   preferred_element_type=jnp.float32)


def _mlp_body(h_ref, p0_ref, p1_ref, w1_ref, b1_ref, w2_ref, b2_ref, o_ref,
              *, relu_out):
    m = h_ref[...] + p0_ref[...] + p1_ref[...]
    y = jnp.maximum(_dot(m, w1_ref[...]) + b1_ref[...], 0.0)
    y = (_dot(y, w2_ref[...]) + b2_ref[...]) * BNSCALE
    if relu_out:
        y = jnp.maximum(y, 0.0)
    o_ref[...] = y


def _mlp(h, p, w1, b1, w2, b2, relu_out):
    blk = 1024
    return pl.pallas_call(
        functools.partial(_mlp_body, relu_out=relu_out),
        grid=(NPAD // blk,),
        in_specs=[
            pl.BlockSpec((blk, H), lambda i: (i, 0)),
            pl.BlockSpec((blk, H), lambda i: (i, 0)),
            pl.BlockSpec((blk, H), lambda i: (i, 0)),
            pl.BlockSpec((H, H), lambda i: (0, 0)),
            pl.BlockSpec((1, H), lambda i: (0, 0)),
            pl.BlockSpec((H, H), lambda i: (0, 0)),
            pl.BlockSpec((1, H), lambda i: (0, 0)),
        ],
        out_specs=pl.BlockSpec((blk, H), lambda i: (i, 0)),
        out_shape=jax.ShapeDtypeStruct((NPAD, H), jnp.float32),
    )(h, p[0], p[1], w1, b1.reshape(1, H), w2, b2.reshape(1, H))


def _score_body(mean_ref, max_ref, add_ref, ord_ref,
                wo1_ref, bo1_ref, wo2_ref, bo2_ref,
                wm_ref, wx_ref, wa_ref, we_ref, bs1_ref,
                ws2_ref, bs2_ref, ws3_ref, bs3_ref, o_ref):
    oe = jnp.maximum(_dot(ord_ref[...], wo1_ref[...]) + bo1_ref[...], 0.0)
    oe = _dot(oe, wo2_ref[...]) + bo2_ref[...]
    pp = (_dot(mean_ref[...], wm_ref[...]) + _dot(max_ref[...], wx_ref[...])
          + _dot(add_ref[...], wa_ref[...]) + bs1_ref[...])
    ppb = jnp.broadcast_to(pp[:, None, :], (B, NO, H)).reshape(B * NO, H)
    s1 = jnp.maximum(_dot(oe, we_ref[...]) + ppb, 0.0)
    s2 = jnp.maximum(_dot(s1, ws2_ref[...]) + bs2_ref[...], 0.0)
    o_ref[...] = _dot(s2, ws3_ref[...]) + bs3_ref[...]


def _score(meanp, maxp, addp, ords, wo1, bo1, wo2, bo2, ws1, bs1,
           ws2, bs2, ws3, bs3):
    ws3p = jnp.zeros((H // 2, H), jnp.float32).at[:, 0].set(ws3[:, 0])
    return pl.pallas_call(
        _score_body,
        out_shape=jax.ShapeDtypeStruct((B * NO, H), jnp.float32),
    )(meanp, maxp, addp, ords,
      wo1, bo1.reshape(1, H), wo2, bo2.reshape(1, H),
      ws1[0:H], ws1[H:2 * H], ws1[2 * H:3 * H], ws1[3 * H:4 * H],
      bs1.reshape(1, H), ws2, bs2.reshape(1, H // 2), ws3p,
      bs3.reshape(1, 1))


def kernel(x, edge_index, edge_attr, orders, batch,
           w1_0, b1_0, w2_0, b2_0,
           w1_1, b1_1, w2_1, b2_1,
           w1_2, b1_2, w2_2, b2_2,
           wo1, bo1, wo2, bo2,
           ws1, bs1, ws2, bs2, ws3, bs3):
    pad = EPAD - E
    srcp = jnp.concatenate(
        [edge_index[0], jnp.zeros((pad,), jnp.int32)]).reshape(NW, NCH, EPC)
    dstp = jnp.concatenate(
        [edge_index[1], jnp.full((pad,), N, jnp.int32)]).reshape(NW, NCH, EPC)
    zeros_blk = jnp.zeros((ZROWS, H), jnp.float32)
    h = jnp.zeros((NPAD, F), jnp.float32).at[:N].set(x)
    batch_p = jnp.concatenate([batch, jnp.full((NB - N,), B, jnp.int32)])

    gin_aggregate, pool = _sc_kernels()
    convs = [(w1_0, b1_0, w2_0, b2_0, True),
             (w1_1, b1_1, w2_1, b2_1, True),
             (w1_2, b1_2, w2_2, b2_2, False)]
    for w1, b1, w2, b2, relu_out in convs:
        p = gin_aggregate(h, srcp, dstp, zeros_blk)
        h = _mlp(h, p, w1, b1, w2, b2, relu_out)

    meanp, maxp, addp = pool(h, batch_p)
    meanp = meanp.reshape(B, H)
    maxp = maxp.reshape(B, H)
    addp = addp.reshape(B, H)
    s3 = _score(meanp, maxp, addp, orders.reshape(B * NO, OD),
                wo1, bo1, wo2, bo2, ws1, bs1, ws2, bs2, ws3, bs3)
    return s3[:, 0].reshape(B, NO)
